# baseline (speedup 1.0000x reference)
import jax
import jax.numpy as jnp
from jax.experimental import pallas as pl
from jax.experimental.pallas import tpu as pltpu

_PARALLEL = pltpu.CompilerParams(dimension_semantics=("parallel",))
_GB = 8


def _fused_kernel(zt_ref, xf_ref, wtT_ref, btT_ref, ws_ref, bs_ref, ab_ref,
                  out_ref):
    for bb in range(_GB):
        zb = zt_ref[bb].astype(jnp.bfloat16)
        xb = xf_ref[bb].astype(jnp.bfloat16)
        acc_t = jnp.zeros((48, 768), jnp.float32)
        acc_s = jnp.zeros((256, 526), jnp.float32)
        for i in range(3):
            for j in range(3):
                tap = i * 3 + j
                zo = i * 8 + j
                xo = i * 24 + j
                acc_t += jnp.dot(zb[zo:zo + 48, :], wtT_ref[tap],
                                 preferred_element_type=jnp.float32)
                acc_s += jnp.dot(ws_ref[tap], xb[:, xo:xo + 526],
                                 preferred_element_type=jnp.float32)
        k3 = (acc_t + btT_ref[...]).astype(jnp.bfloat16).reshape(48, 6, 128)
        sb = (acc_s + bs_ref[...]).astype(jnp.bfloat16)
        sc = sb[0:128, :]
        sl = sb[128:256, :]
        acc_c = jnp.zeros((2, 526), jnp.float32)
        acc_l = jnp.broadcast_to(ab_ref[...], (4, 526)).astype(jnp.float32)
        prev = None
        for di in range(5, -1, -1):
            for dj in range(5, -1, -1):
                off = di * 24 + dj
                if prev is not None:
                    acc_c = jnp.roll(acc_c, off - prev, axis=1)
                    acc_l = jnp.roll(acc_l, off - prev, axis=1)
                t8 = di * 8 + dj
                acc_c = acc_c + jnp.dot(k3[t8, 0:2, :], sc,
                                        preferred_element_type=jnp.float32)
                acc_l = acc_l + jnp.dot(k3[t8, 2:6, :], sl,
                                        preferred_element_type=jnp.float32)
                prev = off
        out_ref[bb, 0:2, :] = acc_c[:, 0:401]
        out_ref[bb, 2:6, :] = acc_l[:, 0:401]


def kernel(z, x, tc_w, tc_b, tl_w, tl_b, sc_w, sc_b, sl_w, sl_b, adj_w, adj_b):
    f32 = jnp.float32
    bf16 = jnp.bfloat16
    B, C = z.shape[0], z.shape[1]

    zt = jnp.pad(z.astype(f32).reshape(B, C, 64).transpose(0, 2, 1),
                 ((0, 0), (0, 8), (0, 0)))
    xf = x.astype(f32).reshape(B, C, 576)
    aw = adj_w[:, :, 0, 0].astype(f32)
    wtl = jnp.einsum('pn,ncr->pcr', aw,
                     tl_w.astype(f32).reshape(4, C, C * 9)).reshape(4 * C, C * 9)
    wt_all = jnp.concatenate([tc_w.astype(f32).reshape(2 * C, C * 9), wtl], 0)
    wtT9 = wt_all.reshape(6 * C, C, 9).transpose(2, 1, 0).astype(bf16)
    btl = (aw @ tl_b.astype(f32).reshape(4, C)).reshape(4 * C)
    btT = jnp.concatenate([tc_b.astype(f32), btl]).reshape(1, 6 * C)
    ws_all = jnp.concatenate([sc_w, sl_w], 0).astype(f32)
    ws9 = ws_all.transpose(2, 3, 0, 1).reshape(9, 2 * C, C).astype(bf16)
    bs = jnp.concatenate([sc_b, sl_b]).astype(f32).reshape(2 * C, 1)
    ab = adj_b.astype(f32).reshape(4, 1)

    out = pl.pallas_call(
        _fused_kernel,
        out_shape=jax.ShapeDtypeStruct((B, 8, 401), f32),
        grid=(B // _GB,),
        in_specs=[
            pl.BlockSpec((_GB, 72, C), lambda b: (b, 0, 0)),
            pl.BlockSpec((_GB, C, 576), lambda b: (b, 0, 0)),
            pl.BlockSpec((9, C, 6 * C), lambda b: (0, 0, 0)),
            pl.BlockSpec((1, 6 * C), lambda b: (0, 0)),
            pl.BlockSpec((9, 2 * C, C), lambda b: (0, 0, 0)),
            pl.BlockSpec((2 * C, 1), lambda b: (0, 0)),
            pl.BlockSpec((4, 1), lambda b: (0, 0)),
        ],
        out_specs=pl.BlockSpec((_GB, 8, 401), lambda b: (b, 0, 0)),
        compiler_params=_PARALLEL,
    )(zt, xf, wtT9, btT, ws9, bs, ab)

    r = jnp.pad(out, ((0, 0), (0, 0), (0, 7))).reshape(B, 8, 17, 24)[:, :, :, :17]
    return r[:, 0:2], r[:, 2:6]


# speedup vs baseline: 1.7894x; 1.7894x over previous
import jax
import jax.numpy as jnp
from jax.experimental import pallas as pl
from jax.experimental.pallas import tpu as pltpu

_PARALLEL = pltpu.CompilerParams(dimension_semantics=("parallel",))
_GB = 8


def _fused_kernel(zt_ref, xf_ref, wt_ref, btT_ref, ws_ref, bs_ref, ab_ref,
                  out_ref):
    bf16 = jnp.bfloat16
    f32 = jnp.float32
    for bb in range(_GB):
        zb = zt_ref[bb].astype(bf16)
        xb = xf_ref[bb].astype(bf16)
        zbig = jnp.concatenate(
            [zb[(i * 8 + j):(i * 8 + j) + 48, :]
             for i in range(3) for j in range(3)], axis=1)
        xbig = jnp.concatenate(
            [xb[:, (i * 24 + j):(i * 24 + j) + 526]
             for i in range(3) for j in range(3)], axis=0)
        t = jnp.dot(zbig, wt_ref[...], preferred_element_type=f32)
        k3 = (t + btT_ref[...]).astype(bf16).reshape(48, 6, 128)
        s = jnp.dot(ws_ref[...], xbig, preferred_element_type=f32)
        sb = (s + bs_ref[...]).astype(bf16)
        s_c = jnp.concatenate([sb[0:128, dj:dj + 521] for dj in range(6)],
                              axis=0)
        s_l = jnp.concatenate([sb[128:256, dj:dj + 521] for dj in range(6)],
                              axis=0)
        k_c = jnp.concatenate(
            [jnp.concatenate([k3[di * 8 + dj, 0:2, :] for dj in range(6)],
                             axis=1) for di in range(6)], axis=0)
        k_l = jnp.concatenate(
            [jnp.concatenate([k3[di * 8 + dj, 2:6, :] for dj in range(6)],
                             axis=1) for di in range(6)], axis=0)
        rc = jnp.dot(k_c, s_c, preferred_element_type=f32)
        rl = jnp.dot(k_l, s_l, preferred_element_type=f32)
        acc_c = rc[10:12, :]
        acc_l = rl[20:24, :]
        for di in range(4, -1, -1):
            acc_c = jnp.roll(acc_c, -24, axis=1) + rc[di * 2:di * 2 + 2, :]
            acc_l = jnp.roll(acc_l, -24, axis=1) + rl[di * 4:di * 4 + 4, :]
        out_ref[bb, 0:2, :] = acc_c[:, 0:401]
        out_ref[bb, 2:6, :] = acc_l[:, 0:401] + ab_ref[...]


def kernel(z, x, tc_w, tc_b, tl_w, tl_b, sc_w, sc_b, sl_w, sl_b, adj_w, adj_b):
    f32 = jnp.float32
    bf16 = jnp.bfloat16
    B, C = z.shape[0], z.shape[1]

    zt = jnp.pad(z.astype(f32).reshape(B, C, 64).transpose(0, 2, 1),
                 ((0, 0), (0, 8), (0, 0)))
    xf = x.astype(f32).reshape(B, C, 576)
    aw = adj_w[:, :, 0, 0].astype(f32)
    wtl = jnp.einsum('pn,ncr->pcr', aw,
                     tl_w.astype(f32).reshape(4, C, C * 9)).reshape(4 * C, C * 9)
    wt_all = jnp.concatenate([tc_w.astype(f32).reshape(2 * C, C * 9), wtl], 0)
    wt_big = (wt_all.reshape(6 * C, C, 9).transpose(2, 1, 0)
              .reshape(9 * C, 6 * C).astype(bf16))
    btl = (aw @ tl_b.astype(f32).reshape(4, C)).reshape(4 * C)
    btT = jnp.concatenate([tc_b.astype(f32), btl]).reshape(1, 6 * C)
    ws_all = jnp.concatenate([sc_w, sl_w], 0).astype(f32)
    ws_big = (ws_all.transpose(2, 3, 1, 0).reshape(9 * C, 2 * C)
              .transpose(1, 0).astype(bf16))
    bs = jnp.concatenate([sc_b, sl_b]).astype(f32).reshape(2 * C, 1)
    ab = adj_b.astype(f32).reshape(4, 1)

    out = pl.pallas_call(
        _fused_kernel,
        out_shape=jax.ShapeDtypeStruct((B, 8, 401), f32),
        grid=(B // _GB,),
        in_specs=[
            pl.BlockSpec((_GB, 72, C), lambda b: (b, 0, 0)),
            pl.BlockSpec((_GB, C, 576), lambda b: (b, 0, 0)),
            pl.BlockSpec((9 * C, 6 * C), lambda b: (0, 0)),
            pl.BlockSpec((1, 6 * C), lambda b: (0, 0)),
            pl.BlockSpec((2 * C, 9 * C), lambda b: (0, 0)),
            pl.BlockSpec((2 * C, 1), lambda b: (0, 0)),
            pl.BlockSpec((4, 1), lambda b: (0, 0)),
        ],
        out_specs=pl.BlockSpec((_GB, 8, 401), lambda b: (b, 0, 0)),
        compiler_params=_PARALLEL,
    )(zt, xf, wt_big, btT, ws_big, bs, ab)

    r = jnp.pad(out, ((0, 0), (0, 0), (0, 7))).reshape(B, 8, 17, 24)[:, :, :, :17]
    return r[:, 0:2], r[:, 2:6]
